# baseline (speedup 1.0000x reference)
import functools
import numpy as np
import jax
import jax.numpy as jnp
from jax import lax
from jax.experimental import pallas as pl
from jax.experimental.pallas import tpu as pltpu
from jax.experimental.pallas import tpu_sc as plsc

_STRIDES = (8, 16, 32)
_IMG = 608
_ANCHORS = np.array(
    [[12, 16], [19, 36], [40, 28], [36, 75], [76, 55], [72, 146],
     [142, 110], [192, 243], [459, 401]], dtype=np.float32)
_NB = 8
_NT = 16
_NREAL = 10
_REC = 10


def _logc(x):
    return jnp.maximum(jnp.log(jnp.maximum(x, 1e-38)), -100.0)


def _bce(p, t):
    return -(t * _logc(p) + (1.0 - t) * _logc(1.0 - p))


def _sig(v):
    return 1.0 / (1.0 + jnp.exp(-v))


def _sc_body(lab_hbm, out_hbm, lab_v, out_v):
    cid = lax.axis_index("c")
    sid = lax.axis_index("s")
    wid = sid * 2 + cid

    @pl.when(wid < 24)
    def _():
        oid = wid // 8
        b = wid - 8 * oid
        s_inv = jnp.where(oid == 0, 1.0 / 8.0,
                          jnp.where(oid == 1, 1.0 / 16.0, 1.0 / 32.0))
        f_max = jnp.where(oid == 0, 75, jnp.where(oid == 1, 37, 18))

        for c in range(5):
            pltpu.sync_copy(lab_hbm.at[c, b], lab_v.at[c])
        lv0 = lab_v[0]
        lv1 = lab_v[1]
        lv2 = lab_v[2]
        lv3 = lab_v[3]
        lv4 = lab_v[4]
        valid = (lv0 + lv1 + lv2 + lv3 + lv4) > 0.0

        tx = (lv0 + lv2) * (0.5 * s_inv)
        ty = (lv1 + lv3) * (0.5 * s_inv)
        tw = (lv2 - lv0) * s_inv
        th = (lv3 - lv1) * s_inv
        area_t = tw * th

        best = jnp.full((16,), -1.0, jnp.float32)
        bestk = jnp.zeros((16,), jnp.int32)
        for k in range(9):
            awk = float(_ANCHORS[k, 0]) * s_inv
            ahk = float(_ANCHORS[k, 1]) * s_inv
            mw = jnp.minimum(tw, awk)
            mh = jnp.minimum(th, ahk)
            ai = mw * mh
            en = (mw > 0.0) & (mh > 0.0)
            iou = jnp.where(en, ai / (area_t + awk * ahk - ai), 0.0)
            upd = iou > best
            best = jnp.where(upd, iou, best)
            bestk = jnp.where(upd, k, bestk)
        cond = valid & (bestk >= 3 * oid) & (bestk < 3 * oid + 3)
        a_i = jnp.minimum(jnp.maximum(bestk - 3 * oid, 0), 2)

        i_i = jnp.minimum(jnp.maximum(tx.astype(jnp.int32), 0), f_max)
        j_i = jnp.minimum(jnp.maximum(ty.astype(jnp.int32), 0), f_max)

        out_v[pl.ds(0, 16)] = a_i.astype(jnp.float32)
        out_v[pl.ds(16, 16)] = i_i.astype(jnp.float32)
        out_v[pl.ds(32, 16)] = j_i.astype(jnp.float32)
        out_v[pl.ds(48, 16)] = jnp.where(cond, 1.0, 0.0)
        out_v[pl.ds(64, 16)] = tx
        out_v[pl.ds(80, 16)] = ty
        out_v[pl.ds(96, 16)] = tw
        out_v[pl.ds(112, 16)] = th
        pltpu.sync_copy(out_v, out_hbm.at[wid])


_sc_match = functools.partial(
    pl.kernel,
    out_type=jax.ShapeDtypeStruct((3 * _NB, _REC * 16), jnp.float32),
    mesh=plsc.VectorSubcoreMesh(core_axis_name="c", subcore_axis_name="s"),
    scratch_types=[
        pltpu.VMEM((5, 16), jnp.float32),
        pltpu.VMEM((_REC * 16,), jnp.float32),
    ],
)(_sc_body)


def _yolo_body(x0_ref, x1_ref, x2_ref, rec_ref, o_ref):
    t_xy = 0.0
    t_wh = 0.0
    t_obj = 0.0
    t_cls = 0.0
    t_l2 = 0.0

    for oid, x_ref in enumerate((x0_ref, x1_ref, x2_ref)):
        s = float(_STRIDES[oid])
        F = _IMG // _STRIDES[oid]
        F2 = F * F
        ma = _ANCHORS[3 * oid:3 * oid + 3] / s

        def _fld(c, oid=oid):
            return rec_ref[pl.ds(oid * _NB, _NB), pl.ds(c * 16, 16)]
        af = _fld(0)
        i_f = _fld(1)
        j_f = _fld(2)
        cond = _fld(3) > 0.5
        tx = _fld(4)
        ty = _fld(5)
        tw = _fld(6)
        th = _fld(7)
        a_i = af.astype(jnp.int32)
        i_i = i_f.astype(jnp.int32)
        j_i = j_f.astype(jnp.int32)
        area_t = tw * th

        key = (a_i * F + j_i) * F + i_i
        tt = lax.broadcasted_iota(jnp.int32, (_NB, _NT, _NT), 1)
        uu = lax.broadcasted_iota(jnp.int32, (_NB, _NT, _NT), 2)
        later_same = ((key[:, :, None] == key[:, None, :])
                      & cond[:, None, :] & (uu > tt))
        winner = cond & jnp.logical_not(jnp.any(later_same, axis=2))
        cond_b = jnp.any(cond, axis=1, keepdims=True)

        tx1 = tx - 0.5 * tw
        tx2 = tx + 0.5 * tw
        ty1 = ty - 0.5 * th
        ty2 = ty + 0.5 * th
        ta3 = area_t * (1.0 / 3.0)

        il = lax.broadcasted_iota(jnp.int32, (_NB, F2), 1)
        iy = (il // F).astype(jnp.float32)
        ix = (il - (il // F) * F).astype(jnp.float32)
        cellf = (j_f * float(F)) + i_f
        cellfull = jnp.concatenate(
            [cellf[b:b + 1, :] for b in range(_NB)], axis=1)
        ohc = (lax.broadcasted_iota(jnp.int32, (F2, _NB * _NT), 0)
               .astype(jnp.float32) == cellfull).astype(jnp.float32)
        rowsel = [
            (lax.broadcasted_iota(jnp.int32, (_NB, _NT), 0) == b)
            .astype(jnp.float32) for b in range(_NB)]
        gval = [jnp.zeros((_NB, _NT), jnp.float32) for _ in range(6)]

        for a in range(3):
            sel_af = [(a_i == aa).astype(jnp.float32) for aa in range(3)]
            o0 = x_ref[:, pl.ds((6 * a + 0) * F2, F2)]
            o1 = x_ref[:, pl.ds((6 * a + 1) * F2, F2)]
            o2 = x_ref[:, pl.ds((6 * a + 2) * F2, F2)]
            o3 = x_ref[:, pl.ds((6 * a + 3) * F2, F2)]
            o4 = x_ref[:, pl.ds((6 * a + 4) * F2, F2)]
            o5 = x_ref[:, pl.ds((6 * a + 5) * F2, F2)]
            s0 = _sig(o0)
            s1 = _sig(o1)
            pw = jnp.exp(o2) * float(ma[a, 0])
            ph = jnp.exp(o3) * float(ma[a, 1])
            px = s0 + ix
            py = s1 + iy
            hx = 0.5 * pw
            hy = 0.5 * ph
            x1p = px - hx
            x2p = px + hx
            y1p = py - hy
            y2p = py + hy
            pa3 = pw * ph * (1.0 / 3.0)
            accm = jnp.full((_NB, F2), -3.0e38, jnp.float32)
            for t in range(_NREAL):
                tx1t = tx1[:, t:t + 1]
                tx2t = tx2[:, t:t + 1]
                ty1t = ty1[:, t:t + 1]
                ty2t = ty2[:, t:t + 1]
                ta3t = ta3[:, t:t + 1]
                dx = jnp.minimum(x2p, tx2t) - jnp.maximum(x1p, tx1t)
                dy = jnp.minimum(y2p, ty2t) - jnp.maximum(y1p, ty1t)
                ai2 = jnp.maximum(dx, 0.0) * jnp.maximum(dy, 0.0)
                accm = jnp.maximum(accm, ai2 - ta3t)
            pbest = accm > pa3
            p4 = _sig(o4)
            om = jnp.where(cond_b, jnp.where(pbest, 0.0, 1.0), 1.0)
            q = p4 * om
            t_obj = t_obj + jnp.sum(-_logc(1.0 - q))
            t_l2 = t_l2 + jnp.sum(q * q)

            for c, oc in enumerate((o0, o1, o2, o3, o4, o5)):
                res = jnp.dot(oc, ohc,
                              preferred_element_type=jnp.float32)
                picked = jnp.zeros((_NB, _NT), jnp.float32)
                for b in range(_NB):
                    picked = picked + (res[:, b * _NT:(b + 1) * _NT]
                                       * rowsel[b])
                gval[c] = gval[c] + picked * sel_af[a]
        g0, g1, g2, g3, g4, g5 = gval

        s0g = _sig(g0)
        s1g = _sig(g1)
        p4g = _sig(g4)
        p5g = _sig(g5)
        aw_sel = jnp.where(a_i == 0, float(ma[0, 0]),
                           jnp.where(a_i == 1, float(ma[1, 0]),
                                     float(ma[2, 0])))
        ah_sel = jnp.where(a_i == 0, float(ma[0, 1]),
                           jnp.where(a_i == 1, float(ma[1, 1]),
                                     float(ma[2, 1])))
        pxc = s0g + i_f
        pyc = s1g + j_f
        pwc = jnp.exp(g2) * aw_sel
        phc = jnp.exp(g3) * ah_sel
        hxc = 0.5 * pwc
        hyc = 0.5 * phc
        dxu = (jnp.minimum((pxc + hxc)[:, :, None], tx2[:, None, :_NREAL])
               - jnp.maximum((pxc - hxc)[:, :, None], tx1[:, None, :_NREAL]))
        dyu = (jnp.minimum((pyc + hyc)[:, :, None], ty2[:, None, :_NREAL])
               - jnp.maximum((pyc - hyc)[:, :, None], ty1[:, None, :_NREAL]))
        aiu = jnp.maximum(dxu, 0.0) * jnp.maximum(dyu, 0.0)
        mu = jnp.max(aiu - ta3[:, None, :_NREAL], axis=2)
        pbc = mu > pwc * phc * (1.0 / 3.0)
        omc = jnp.where(cond_b, jnp.where(pbc, 0.0, 1.0), 1.0)

        tg0 = tx - i_f
        tg1 = ty - j_f
        tg2 = jnp.log(tw / aw_sel + 1e-16)
        tg3 = jnp.log(th / ah_sel + 1e-16)
        scv = jnp.sqrt(2.0 - area_t * (1.0 / (F * F)))
        w = winner.astype(jnp.float32)

        dxy = (_bce(s0g, tg0) + _bce(s1g, tg1)) * (scv * scv)
        dwh = ((g2 * scv - tg2 * scv) ** 2 + (g3 * scv - tg3 * scv) ** 2) * 0.5
        qg = p4g * omc
        dobj = -_logc(p4g) - (-_logc(1.0 - qg))
        dcls = -_logc(p5g)
        dl2 = ((s0g - tg0) ** 2 + (s1g - tg1) ** 2
               + (g2 * scv - tg2 * scv) ** 2 + (g3 * scv - tg3 * scv) ** 2
               + (p4g - 1.0) ** 2 + (p5g - 1.0) ** 2 - qg * qg)

        t_xy = t_xy + jnp.sum(w * dxy)
        t_wh = t_wh + jnp.sum(w * dwh)
        t_obj = t_obj + jnp.sum(w * dobj)
        t_cls = t_cls + jnp.sum(w * dcls)
        t_l2 = t_l2 + jnp.sum(w * dl2)

    o_ref[0, 0] = t_xy + t_wh + t_obj + t_cls
    o_ref[0, 1] = t_xy
    o_ref[0, 2] = t_wh
    o_ref[0, 3] = t_obj
    o_ref[0, 4] = t_cls
    o_ref[0, 5] = t_l2


def kernel(x0, x1, x2, labels):
    labT = jnp.transpose(labels[:, :_NT, :], (2, 0, 1))
    rec = _sc_match(labT)
    xp = [x.reshape(_NB, -1) for x in (x0, x1, x2)]
    out = pl.pallas_call(
        _yolo_body,
        out_shape=jax.ShapeDtypeStruct((1, 8), jnp.float32),
        out_specs=pl.BlockSpec(memory_space=pltpu.SMEM),
        in_specs=[pl.BlockSpec(memory_space=pltpu.VMEM)] * 4,
    )(xp[0], xp[1], xp[2], rec)
    return (out[0, 0], out[0, 1], out[0, 2], out[0, 3], out[0, 4], out[0, 5])


# speedup vs baseline: 1.1250x; 1.1250x over previous
import functools
import numpy as np
import jax
import jax.numpy as jnp
from jax import lax
from jax.experimental import pallas as pl
from jax.experimental.pallas import tpu as pltpu
from jax.experimental.pallas import tpu_sc as plsc

_STRIDES = (8, 16, 32)
_IMG = 608
_ANCHORS = np.array(
    [[12, 16], [19, 36], [40, 28], [36, 75], [76, 55], [72, 146],
     [142, 110], [192, 243], [459, 401]], dtype=np.float32)
_NB = 8
_NT = 16
_NREAL = 10
_REC = 10


def _logc(x):
    return jnp.maximum(jnp.log(jnp.maximum(x, 1e-38)), -100.0)


def _bce(p, t):
    return -(t * _logc(p) + (1.0 - t) * _logc(1.0 - p))


def _sig(v):
    return 1.0 / (1.0 + jnp.exp(-v))


def _sc_body(lab_hbm, out_hbm, lab_v, out_v):
    cid = lax.axis_index("c")
    sid = lax.axis_index("s")
    wid = sid * 2 + cid

    @pl.when(wid < 24)
    def _():
        oid = wid // 8
        b = wid - 8 * oid
        s_inv = jnp.where(oid == 0, 1.0 / 8.0,
                          jnp.where(oid == 1, 1.0 / 16.0, 1.0 / 32.0))
        f_max = jnp.where(oid == 0, 75, jnp.where(oid == 1, 37, 18))

        for c in range(5):
            pltpu.sync_copy(lab_hbm.at[c, b], lab_v.at[c])
        lv0 = lab_v[0]
        lv1 = lab_v[1]
        lv2 = lab_v[2]
        lv3 = lab_v[3]
        lv4 = lab_v[4]
        valid = (lv0 + lv1 + lv2 + lv3 + lv4) > 0.0

        tx = (lv0 + lv2) * (0.5 * s_inv)
        ty = (lv1 + lv3) * (0.5 * s_inv)
        tw = (lv2 - lv0) * s_inv
        th = (lv3 - lv1) * s_inv
        area_t = tw * th

        best = jnp.full((16,), -1.0, jnp.float32)
        bestk = jnp.zeros((16,), jnp.int32)
        for k in range(9):
            awk = float(_ANCHORS[k, 0]) * s_inv
            ahk = float(_ANCHORS[k, 1]) * s_inv
            mw = jnp.minimum(tw, awk)
            mh = jnp.minimum(th, ahk)
            ai = mw * mh
            en = (mw > 0.0) & (mh > 0.0)
            iou = jnp.where(en, ai / (area_t + awk * ahk - ai), 0.0)
            upd = iou > best
            best = jnp.where(upd, iou, best)
            bestk = jnp.where(upd, k, bestk)
        cond = valid & (bestk >= 3 * oid) & (bestk < 3 * oid + 3)
        a_i = jnp.minimum(jnp.maximum(bestk - 3 * oid, 0), 2)

        i_i = jnp.minimum(jnp.maximum(tx.astype(jnp.int32), 0), f_max)
        j_i = jnp.minimum(jnp.maximum(ty.astype(jnp.int32), 0), f_max)

        out_v[pl.ds(0, 16)] = a_i.astype(jnp.float32)
        out_v[pl.ds(16, 16)] = i_i.astype(jnp.float32)
        out_v[pl.ds(32, 16)] = j_i.astype(jnp.float32)
        out_v[pl.ds(48, 16)] = jnp.where(cond, 1.0, 0.0)
        out_v[pl.ds(64, 16)] = tx
        out_v[pl.ds(80, 16)] = ty
        out_v[pl.ds(96, 16)] = tw
        out_v[pl.ds(112, 16)] = th
        pltpu.sync_copy(out_v, out_hbm.at[wid])


_sc_match = functools.partial(
    pl.kernel,
    out_type=jax.ShapeDtypeStruct((3 * _NB, _REC * 16), jnp.float32),
    mesh=plsc.VectorSubcoreMesh(core_axis_name="c", subcore_axis_name="s"),
    scratch_types=[
        pltpu.VMEM((5, 16), jnp.float32),
        pltpu.VMEM((_REC * 16,), jnp.float32),
    ],
)(_sc_body)


def _yolo_body(x0_ref, x1_ref, x2_ref, rec_ref, o_ref):
    t_xy = 0.0
    t_wh = 0.0
    t_obj = 0.0
    t_cls = 0.0
    t_l2 = 0.0

    for oid, x_ref in enumerate((x0_ref, x1_ref, x2_ref)):
        s = float(_STRIDES[oid])
        F = _IMG // _STRIDES[oid]
        F2 = F * F
        ma = _ANCHORS[3 * oid:3 * oid + 3] / s

        def _fld(c, oid=oid):
            return rec_ref[pl.ds(oid * _NB, _NB), pl.ds(c * 16, 16)]
        af = _fld(0)
        i_f = _fld(1)
        j_f = _fld(2)
        cond = _fld(3) > 0.5
        tx = _fld(4)
        ty = _fld(5)
        tw = _fld(6)
        th = _fld(7)
        a_i = af.astype(jnp.int32)
        i_i = i_f.astype(jnp.int32)
        j_i = j_f.astype(jnp.int32)
        area_t = tw * th

        key = (a_i * F + j_i) * F + i_i
        tt = lax.broadcasted_iota(jnp.int32, (_NB, _NT, _NT), 1)
        uu = lax.broadcasted_iota(jnp.int32, (_NB, _NT, _NT), 2)
        later_same = ((key[:, :, None] == key[:, None, :])
                      & cond[:, None, :] & (uu > tt))
        winner = cond & jnp.logical_not(jnp.any(later_same, axis=2))
        cond_b = jnp.any(cond, axis=1, keepdims=True)

        tx1 = tx - 0.5 * tw
        tx2 = tx + 0.5 * tw
        ty1 = ty - 0.5 * th
        ty2 = ty + 0.5 * th
        ta3 = area_t * (1.0 / 3.0)

        il = lax.broadcasted_iota(jnp.int32, (_NB, F2), 1)
        iy = (il // F).astype(jnp.float32)
        ix = (il - (il // F) * F).astype(jnp.float32)
        iotaC = lax.broadcasted_iota(jnp.int32, (F2, _NT), 0).astype(
            jnp.float32)
        cellf = (j_f * float(F)) + i_f

        for a in range(3):
            o0 = x_ref[6 * a + 0]
            o1 = x_ref[6 * a + 1]
            o2 = x_ref[6 * a + 2]
            o3 = x_ref[6 * a + 3]
            o4 = x_ref[6 * a + 4]
            s0 = _sig(o0)
            s1 = _sig(o1)
            pw = jnp.exp(o2) * float(ma[a, 0])
            ph = jnp.exp(o3) * float(ma[a, 1])
            px = s0 + ix
            py = s1 + iy
            hx = 0.5 * pw
            hy = 0.5 * ph
            x1p = px - hx
            x2p = px + hx
            y1p = py - hy
            y2p = py + hy
            pa3 = pw * ph * (1.0 / 3.0)
            accm = jnp.full((_NB, F2), -3.0e38, jnp.float32)
            for t in range(_NREAL):
                tx1t = tx1[:, t:t + 1]
                tx2t = tx2[:, t:t + 1]
                ty1t = ty1[:, t:t + 1]
                ty2t = ty2[:, t:t + 1]
                ta3t = ta3[:, t:t + 1]
                dx = jnp.minimum(x2p, tx2t) - jnp.maximum(x1p, tx1t)
                dy = jnp.minimum(y2p, ty2t) - jnp.maximum(y1p, ty1t)
                ai2 = jnp.maximum(dx, 0.0) * jnp.maximum(dy, 0.0)
                accm = jnp.maximum(accm, ai2 - ta3t)
            pbest = accm > pa3
            p4 = _sig(o4)
            om = jnp.where(cond_b, jnp.where(pbest, 0.0, 1.0), 1.0)
            q = p4 * om
            t_obj = t_obj + jnp.sum(-_logc(1.0 - q))
            t_l2 = t_l2 + jnp.sum(q * q)

        g_rows = []
        for b in range(_NB):
            ohc = (iotaC == cellf[b:b + 1, :]).astype(jnp.float32)
            x18b = x_ref[:, b, :]
            red = jnp.dot(x18b, ohc,
                          preferred_element_type=jnp.float32)
            arow = af[b:b + 1, :]
            sel = jnp.concatenate(
                [(arow == 0.0).astype(jnp.float32),
                 (arow == 1.0).astype(jnp.float32),
                 (arow == 2.0).astype(jnp.float32)], axis=0)
            gb = jnp.sum(red.reshape(3, 6, _NT) * sel[:, None, :], axis=0)
            g_rows.append(gb[None])
        gall = jnp.concatenate(g_rows, axis=0)
        g0 = gall[:, 0, :]
        g1 = gall[:, 1, :]
        g2 = gall[:, 2, :]
        g3 = gall[:, 3, :]
        g4 = gall[:, 4, :]
        g5 = gall[:, 5, :]

        s0g = _sig(g0)
        s1g = _sig(g1)
        p4g = _sig(g4)
        p5g = _sig(g5)
        aw_sel = jnp.where(a_i == 0, float(ma[0, 0]),
                           jnp.where(a_i == 1, float(ma[1, 0]),
                                     float(ma[2, 0])))
        ah_sel = jnp.where(a_i == 0, float(ma[0, 1]),
                           jnp.where(a_i == 1, float(ma[1, 1]),
                                     float(ma[2, 1])))
        pxc = s0g + i_f
        pyc = s1g + j_f
        pwc = jnp.exp(g2) * aw_sel
        phc = jnp.exp(g3) * ah_sel
        hxc = 0.5 * pwc
        hyc = 0.5 * phc
        dxu = (jnp.minimum((pxc + hxc)[:, :, None], tx2[:, None, :_NREAL])
               - jnp.maximum((pxc - hxc)[:, :, None], tx1[:, None, :_NREAL]))
        dyu = (jnp.minimum((pyc + hyc)[:, :, None], ty2[:, None, :_NREAL])
               - jnp.maximum((pyc - hyc)[:, :, None], ty1[:, None, :_NREAL]))
        aiu = jnp.maximum(dxu, 0.0) * jnp.maximum(dyu, 0.0)
        mu = jnp.max(aiu - ta3[:, None, :_NREAL], axis=2)
        pbc = mu > pwc * phc * (1.0 / 3.0)
        omc = jnp.where(cond_b, jnp.where(pbc, 0.0, 1.0), 1.0)

        tg0 = tx - i_f
        tg1 = ty - j_f
        tg2 = jnp.log(tw / aw_sel + 1e-16)
        tg3 = jnp.log(th / ah_sel + 1e-16)
        scv = jnp.sqrt(2.0 - area_t * (1.0 / (F * F)))
        w = winner.astype(jnp.float32)

        dxy = (_bce(s0g, tg0) + _bce(s1g, tg1)) * (scv * scv)
        dwh = ((g2 * scv - tg2 * scv) ** 2 + (g3 * scv - tg3 * scv) ** 2) * 0.5
        qg = p4g * omc
        dobj = -_logc(p4g) - (-_logc(1.0 - qg))
        dcls = -_logc(p5g)
        dl2 = ((s0g - tg0) ** 2 + (s1g - tg1) ** 2
               + (g2 * scv - tg2 * scv) ** 2 + (g3 * scv - tg3 * scv) ** 2
               + (p4g - 1.0) ** 2 + (p5g - 1.0) ** 2 - qg * qg)

        t_xy = t_xy + jnp.sum(w * dxy)
        t_wh = t_wh + jnp.sum(w * dwh)
        t_obj = t_obj + jnp.sum(w * dobj)
        t_cls = t_cls + jnp.sum(w * dcls)
        t_l2 = t_l2 + jnp.sum(w * dl2)

    o_ref[0, 0] = t_xy + t_wh + t_obj + t_cls
    o_ref[0, 1] = t_xy
    o_ref[0, 2] = t_wh
    o_ref[0, 3] = t_obj
    o_ref[0, 4] = t_cls
    o_ref[0, 5] = t_l2


def kernel(x0, x1, x2, labels):
    labT = jnp.transpose(labels[:, :_NT, :], (2, 0, 1))
    rec = _sc_match(labT)
    xp = [jnp.transpose(x.reshape(_NB, 18, -1), (1, 0, 2))
          for x in (x0, x1, x2)]
    out = pl.pallas_call(
        _yolo_body,
        out_shape=jax.ShapeDtypeStruct((1, 8), jnp.float32),
        out_specs=pl.BlockSpec(memory_space=pltpu.SMEM),
        in_specs=[pl.BlockSpec(memory_space=pltpu.VMEM)] * 4,
    )(xp[0], xp[1], xp[2], rec)
    return (out[0, 0], out[0, 1], out[0, 2], out[0, 3], out[0, 4], out[0, 5])
